# Initial kernel scaffold; baseline (speedup 1.0000x reference)
import jax
import jax.numpy as jnp
from jax.experimental import pallas as pl


def kernel(pt, w, b):
    raise NotImplementedError("write your pallas kernel here")


# speedup vs baseline: 466.3120x; 466.3120x over previous
import functools

import numpy as np
import jax
import jax.numpy as jnp
from jax import lax
from jax.experimental import pallas as pl
from jax.experimental.pallas import tpu as pltpu
from jax.experimental.pallas import tpu_sc as plsc

_RES = 256
_FEAT = 16
_L = 6
_B = 65536


def _gaussian_kernel(m, std):
    n = np.arange(m, dtype=np.float64) - (m - 1) / 2.0
    k = np.exp(-0.5 * (n / std) ** 2)
    return (k / k.sum()).astype(np.float32).astype(np.float64)


def _blur_matrix(s):
    kern = _gaussian_kernel(s, s / 2.0)
    eye = np.eye(_RES, dtype=np.float64)
    padded = np.pad(eye, ((s, s), (0, 0)), mode="reflect")
    out = np.empty_like(padded)
    for j in range(_RES):
        out[:, j] = np.convolve(padded[:, j], kern, mode="same")
    return out[s:s + _RES, :]


_A_LIST = [_blur_matrix(2 ** i).astype(np.float32) for i in range(1, _L)]
_A_STACK_NP = np.stack(_A_LIST).astype(np.float32)
_AT_NP = np.stack([a.T for a in _A_LIST]).astype(np.float32)

_E_NP = np.zeros((8, _FEAT, 128), dtype=np.float32)
for _j in range(8):
    for _f in range(_FEAT):
        _E_NP[_j, _f, _j * _FEAT + _f] = 1.0


_TROWS = _RES * _RES * _FEAT // 128


def _tc_table_body(b_ref, a_ref, at_ref, e_ref, w_ref, out_ref):
    w_all = w_ref[...]
    w2 = w_all.reshape(_FEAT * _RES, _RES)
    cols = [b_ref[0] * w_all[f] for f in range(_FEAT)]
    for i in range(_L - 1):
        r = jnp.dot(w2, at_ref[i],
                    preferred_element_type=jnp.float32) * b_ref[i + 1]
        r3 = r.reshape(_FEAT, _RES, _RES)
        cols = [cols[f] + jnp.dot(a_ref[i], r3[f],
                                  preferred_element_type=jnp.float32)
                for f in range(_FEAT)]
    t = jnp.stack(cols, axis=1)
    tt = jnp.transpose(t, (0, 2, 1))
    t4 = tt.reshape(_RES, _RES // 8, 8, _FEAT)
    out = jnp.dot(t4[:, :, 0, :].reshape(-1, _FEAT), e_ref[0],
                  preferred_element_type=jnp.float32)
    for j in range(1, 8):
        out = out + jnp.dot(t4[:, :, j, :].reshape(-1, _FEAT), e_ref[j],
                            preferred_element_type=jnp.float32)
    out_ref[...] = out


def _tc_table(b, a, at, e, w_f):
    return pl.pallas_call(
        _tc_table_body,
        in_specs=[
            pl.BlockSpec(memory_space=pltpu.SMEM),
            pl.BlockSpec(((_L - 1), _RES, _RES), lambda: (0, 0, 0)),
            pl.BlockSpec(((_L - 1), _RES, _RES), lambda: (0, 0, 0)),
            pl.BlockSpec((8, _FEAT, 128), lambda: (0, 0, 0)),
            pl.BlockSpec((_FEAT, _RES, _RES), lambda: (0, 0, 0)),
        ],
        out_specs=pl.BlockSpec((_TROWS, 128), lambda: (0, 0)),
        out_shape=jax.ShapeDtypeStruct((_TROWS, 128), jnp.float32),
        compiler_params=pltpu.CompilerParams(
            vmem_limit_bytes=48 * 1024 * 1024),
    )(b, a, at, e, w_f)


_NC, _NS = 2, 16
_NW = _NC * _NS
_BPW = _B // _NW
_CH = 1024
_NCHUNK = _BPW // _CH

_sc_mesh = plsc.VectorSubcoreMesh(
    core_axis_name="c", subcore_axis_name="s", num_cores=_NC, num_subcores=_NS)


@functools.partial(
    pl.kernel,
    out_type=jax.ShapeDtypeStruct((_FEAT, _B), jnp.float32),
    mesh=_sc_mesh,
    compiler_params=pltpu.CompilerParams(use_tc_tiling_on_sc=False,
                                         needs_layout_passes=False),
    scratch_types=[
        pltpu.VMEM((_BPW,), jnp.float32),
        pltpu.VMEM((_BPW,), jnp.float32),
        [pltpu.VMEM((_CH,), jnp.int32) for _ in range(4)],
        [pltpu.VMEM((_CH,), jnp.float32) for _ in range(4)],
        [pltpu.VMEM((_CH, _FEAT), jnp.float32) for _ in range(4)],
        pltpu.VMEM((_FEAT * (_CH + 8),), jnp.float32),
        pltpu.SemaphoreType.DMA,
        pltpu.SemaphoreType.DMA,
    ],
)
def _sc_gather(g_hbm, px_hbm, py_hbm, out_hbm,
               px_v, py_v, idx_v, wt_v, rows_v, out_v, gsem, osem):
    wid = lax.axis_index("s") * _NC + lax.axis_index("c")
    base = wid * _BPW
    pltpu.sync_copy(px_hbm.at[pl.ds(base, _BPW)], px_v)
    pltpu.sync_copy(py_hbm.at[pl.ds(base, _BPW)], py_v)
    lrow = lax.iota(jnp.int32, 16) * (_CH + 8)
    pending_out = []

    for c in range(_NCHUNK):
        off = c * _CH

        def idx_body(j, _):
            src = pl.ds(off + j * 16, 16)
            px = px_v[src]
            py = py_v[src]
            ax = (px + 1.0) * 0.5 * 255.0
            ay = (py + 1.0) * 0.5 * 255.0
            ix = ax.astype(jnp.int32)
            iy = ay.astype(jnp.int32)
            fx = ax - ix.astype(jnp.float32)
            fy = ay - iy.astype(jnp.float32)
            x0 = jnp.clip(ix, 0, _RES - 1) * _RES
            x1 = jnp.clip(ix + 1, 0, _RES - 1) * _RES
            y0 = jnp.clip(iy, 0, _RES - 1)
            y1 = jnp.clip(iy + 1, 0, _RES - 1)
            dst = pl.ds(j * 16, 16)
            idx_v[0][dst] = x0 + y0
            idx_v[1][dst] = x0 + y1
            idx_v[2][dst] = x1 + y0
            idx_v[3][dst] = x1 + y1
            gx = 1.0 - fx
            gy = 1.0 - fy
            wt_v[0][dst] = gx * gy
            wt_v[1][dst] = gx * fy
            wt_v[2][dst] = fx * gy
            wt_v[3][dst] = fx * fy
            return 0

        lax.fori_loop(0, _CH // 16, idx_body, 0)

        copies = [pltpu.async_copy(g_hbm.at[idx_v[k]], rows_v[k], gsem)
                  for k in range(4)]
        for cp in copies:
            cp.wait()
        for cp in pending_out:
            cp.wait()
        pending_out = []

        def combine_body(g, _):
            sl = pl.ds(g * 16, 16)
            w0 = wt_v[0][sl]
            w1 = wt_v[1][sl]
            w2 = wt_v[2][sl]
            w3 = wt_v[3][sl]
            for l in range(16):
                p = g * 16 + l
                acc = ((w0[l] * rows_v[0][p] + w1[l] * rows_v[1][p])
                       + (w2[l] * rows_v[2][p] + w3[l] * rows_v[3][p]))
                plsc.store_scatter(out_v, [lrow + p], acc)
            return 0

        lax.fori_loop(0, _CH // 16, combine_body, 0)
        pending_out = [
            pltpu.async_copy(out_v.at[pl.ds(f * (_CH + 8), _CH)],
                             out_hbm.at[f, pl.ds(base + off, _CH)], osem)
            for f in range(_FEAT)]
    for cp in pending_out:
        cp.wait()


def kernel(pt, w, b):
    a = jnp.asarray(_A_STACK_NP)
    at = jnp.asarray(_AT_NP)
    e = jnp.asarray(_E_NP)
    w_f = jnp.transpose(w, (2, 0, 1))
    table = _tc_table(b, a, at, e, w_f)
    gg = jnp.reshape(table, (_B, _FEAT))
    px = pt[:, 0]
    py = pt[:, 1]
    out_fmaj = _sc_gather(gg, px, py)
    return jnp.transpose(out_fmaj)
